# baseline (speedup 1.0000x reference)
import dataclasses
import functools

import jax
import jax.numpy as jnp
from jax import lax
from jax.experimental import pallas as pl
from jax.experimental.pallas import tpu as pltpu
from jax.experimental.pallas import tpu_sc as plsc

NC = 2
NS = 16
LANES = 16
CB = 80


def _sc_compiler_params():
    cp = pltpu.CompilerParams()
    cp = dataclasses.replace(
        cp, needs_layout_passes=False, use_tc_tiling_on_sc=False,
        internal_scratch_in_bytes=256 * 1024,
    )
    return cp


def _mesh():
    return plsc.VectorSubcoreMesh(
        core_axis_name="c", subcore_axis_name="s", num_cores=NC, num_subcores=NS
    )


def _zero_acc(zsrc, acc, sid, RB, TAIL, ZR):
    for t in range(RB // ZR):
        pltpu.sync_copy(zsrc, acc.at[pl.ds(sid * RB + t * ZR, ZR)])
    if TAIL:
        @pl.when(sid == NS - 1)
        def _():
            pltpu.sync_copy(zsrc.at[pl.ds(0, TAIL)], acc.at[pl.ds(NS * RB, TAIL)])


def _write_out(acc, out_hbm, cid, sid, RB, TAIL):
    pltpu.sync_copy(
        acc.at[pl.ds(sid * RB, RB)], out_hbm.at[cid, pl.ds(sid * RB, RB)]
    )
    if TAIL:
        @pl.when(sid == NS - 1)
        def _():
            pltpu.sync_copy(acc.at[pl.ds(NS * RB, TAIL)],
                            out_hbm.at[cid, pl.ds(NS * RB, TAIL)])


def _deg_kernel(E, N):
    NW = NC * NS
    EPW = E // NW
    NCH = EPW // CB
    RB = (N // NS) & ~7
    TAIL = N - RB * NS
    ZR = 208

    @functools.partial(
        pl.kernel,
        mesh=_mesh(),
        out_type=jax.ShapeDtypeStruct((NC, N, LANES), jnp.float32),
        compiler_params=_sc_compiler_params(),
        scratch_types=[
            pltpu.VMEM((NCH, CB), jnp.int32),
            pltpu.VMEM((EPW,), jnp.float32),
            pltpu.VMEM((CB, LANES), jnp.float32),
            pltpu.VMEM((CB, LANES), jnp.float32),
            pltpu.VMEM((ZR, LANES), jnp.float32),
            pltpu.VMEM_SHARED((N, LANES), jnp.float32),
            pltpu.SemaphoreType.DMA,
            pltpu.SemaphoreType.DMA,
        ],
    )
    def k(dst2_hbm, w_hbm, out_hbm, didx, wbuf, w0, w1, zsrc, acc, s0, s1):
        cid = lax.axis_index("c")
        sid = lax.axis_index("s")
        wid = cid * NS + sid

        @pl.loop(0, ZR)
        def _(r):
            zsrc[r, :] = jnp.zeros((LANES,), jnp.float32)

        _zero_acc(zsrc, acc, sid, RB, TAIL, ZR)
        pltpu.sync_copy(dst2_hbm.at[pl.ds(wid * NCH, NCH)], didx)
        pltpu.sync_copy(w_hbm.at[pl.ds(wid * EPW, EPW)], wbuf)
        plsc.subcore_barrier()

        wrows = (w0, w1)
        sems = (s0, s1)

        def build(j, wr):
            @plsc.parallel_loop(0, CB, unroll=8)
            def _(e):
                wv = plsc.load_gather(
                    wbuf, [jnp.full((LANES,), j * CB + e, jnp.int32)]
                )
                wr[e, :] = wv

        @pl.loop(0, NCH)
        def _(j):
            for b in range(2):
                @pl.when(j % 2 == b)
                def _():
                    @pl.when(j >= 2)
                    def _():
                        pltpu.make_async_copy(
                            wrows[b], acc.at[didx.at[j]], sems[b]
                        ).wait()
                    build(j, wrows[b])
                    pltpu.async_copy(
                        wrows[b], acc.at[didx.at[j]], sems[b], add=True
                    )

        for b in range(2):
            @pl.when(NCH > 2 - b)
            def _():
                pltpu.make_async_copy(
                    wrows[b], acc.at[didx.at[0]], sems[b]
                ).wait()

        plsc.subcore_barrier()
        _write_out(acc, out_hbm, cid, sid, RB, TAIL)

    return k


def _edge_kernel(E, N, D):
    NW = NC * NS
    EPW = E // NW
    NCH = EPW // CB
    RB = (N // NS) & ~7
    TAIL = N - RB * NS
    ZR = 208

    @functools.partial(
        pl.kernel,
        mesh=_mesh(),
        out_type=jax.ShapeDtypeStruct((NC, N, D), jnp.float32),
        compiler_params=_sc_compiler_params(),
        scratch_types=[
            pltpu.VMEM((NCH, CB), jnp.int32),
            pltpu.VMEM((NCH, CB), jnp.int32),
            pltpu.VMEM((EPW,), jnp.float32),
            pltpu.VMEM((CB, D), jnp.float32),
            pltpu.VMEM((CB, D), jnp.float32),
            pltpu.VMEM((CB, D), jnp.float32),
            pltpu.VMEM((ZR, D), jnp.float32),
            pltpu.VMEM_SHARED((N, D), jnp.float32),
            pltpu.SemaphoreType.DMA,
            pltpu.SemaphoreType.DMA,
            pltpu.SemaphoreType.DMA,
            pltpu.SemaphoreType.DMA,
            pltpu.SemaphoreType.DMA,
            pltpu.SemaphoreType.DMA,
        ],
    )
    def k(g_hbm, src2_hbm, dst2_hbm, w_hbm, out_hbm,
          sidx, didx, wbuf, r0, r1, r2, zsrc, acc, g0, g1, g2, s0, s1, s2):
        cid = lax.axis_index("c")
        sid = lax.axis_index("s")
        wid = cid * NS + sid

        @pl.loop(0, ZR)
        def _(r):
            for c4 in range(D // LANES):
                zsrc[r, pl.ds(c4 * LANES, LANES)] = jnp.zeros((LANES,), jnp.float32)

        _zero_acc(zsrc, acc, sid, RB, TAIL, ZR)
        pltpu.sync_copy(src2_hbm.at[pl.ds(wid * NCH, NCH)], sidx)
        pltpu.sync_copy(dst2_hbm.at[pl.ds(wid * NCH, NCH)], didx)
        pltpu.sync_copy(w_hbm.at[pl.ds(wid * EPW, EPW)], wbuf)
        plsc.subcore_barrier()

        rows = (r0, r1, r2)
        gsem = (g0, g1, g2)
        ssem = (s0, s1, s2)

        def compute(j, rr):
            @plsc.parallel_loop(0, CB, unroll=8)
            def _(e):
                wv = plsc.load_gather(
                    wbuf, [jnp.full((LANES,), j * CB + e, jnp.int32)]
                )
                for c4 in range(D // LANES):
                    sl = pl.ds(c4 * LANES, LANES)
                    rr[e, sl] = rr[e, sl] * wv

        pltpu.async_copy(g_hbm.at[sidx.at[0]], r0, g0)
        if NCH > 1:
            pltpu.async_copy(g_hbm.at[sidx.at[1]], r1, g1)

        @pl.loop(0, NCH)
        def _(j):
            for b in range(3):
                b2 = (b + 2) % 3

                @pl.when(j % 3 == b)
                def _():
                    @pl.when(j >= 1)
                    def _():
                        pltpu.make_async_copy(
                            rows[b2], acc.at[didx.at[j]], ssem[b2]
                        ).wait()

                    @pl.when(j + 2 < NCH)
                    def _():
                        pltpu.async_copy(
                            g_hbm.at[sidx.at[j + 2]], rows[b2], gsem[b2]
                        )

                    pltpu.make_async_copy(
                        g_hbm.at[sidx.at[j]], rows[b], gsem[b]
                    ).wait()
                    compute(j, rows[b])
                    pltpu.async_copy(
                        rows[b], acc.at[didx.at[j]], ssem[b], add=True
                    )

        pltpu.make_async_copy(
            rows[(NCH - 1) % 3], acc.at[didx.at[0]], ssem[(NCH - 1) % 3]
        ).wait()

        plsc.subcore_barrier()
        _write_out(acc, out_hbm, cid, sid, RB, TAIL)

    return k


_DOT = dict(preferred_element_type=jnp.float32, precision=lax.Precision.HIGHEST)


def _tc_h1_dis_g(x, W1, deg_part):
    N = x.shape[0]
    H = W1.shape[0]

    def body(x_ref, w_ref, dp_ref, dis_ref, g_ref):
        h = lax.dot_general(
            x_ref[...], w_ref[...], (((1,), (1,)), ((), ())), **_DOT
        )
        deg = dp_ref[0] + dp_ref[1]
        dis = lax.rsqrt(deg[:, 0:1] + 1.0)
        dis_ref[...] = dis
        g_ref[...] = h * dis

    return pl.pallas_call(
        body,
        out_shape=(
            jax.ShapeDtypeStruct((N, 1), jnp.float32),
            jax.ShapeDtypeStruct((N, H), jnp.float32),
        ),
    )(x, W1, deg_part)


def _tc_layer_out(s_part, g, dis, b, W_next, scale_out):
    N, D = g.shape
    H = W_next.shape[0]

    def body(sp_ref, g_ref, dis_ref, b_ref, w_ref, o_ref):
        s = sp_ref[0] + sp_ref[1] + g_ref[...]
        xx = jax.nn.relu(dis_ref[...] * s + b_ref[...])
        h = lax.dot_general(xx, w_ref[...], (((1,), (1,)), ((), ())), **_DOT)
        if scale_out:
            h = h * dis_ref[...]
        o_ref[...] = h

    return pl.pallas_call(
        body, out_shape=jax.ShapeDtypeStruct((N, H), jnp.float32)
    )(s_part, g, dis, b.reshape(1, D), W_next)


def _tc_head(s_part, g, dis, b, Wfc, bfc):
    N, D = g.shape
    O = Wfc.shape[0]

    def body(sp_ref, g_ref, dis_ref, b_ref, w_ref, bfc_ref, o_ref):
        s = sp_ref[0] + sp_ref[1] + g_ref[...]
        xx = jax.nn.relu(dis_ref[...] * s + b_ref[...])
        o_ref[...] = (
            lax.dot_general(xx, w_ref[...], (((1,), (1,)), ((), ())), **_DOT)
            + bfc_ref[...]
        )

    return pl.pallas_call(
        body, out_shape=jax.ShapeDtypeStruct((N, O), jnp.float32)
    )(s_part, g, dis, b.reshape(1, D), Wfc, bfc.reshape(1, O))


def kernel(x, edge_index, edge_attr, W1, b1, W2, b2, Wfc, bfc):
    N = x.shape[0]
    E = edge_index.shape[1]
    H = W1.shape[0]

    src2 = edge_index[0].astype(jnp.int32).reshape(E // CB, CB)
    dst2 = edge_index[1].astype(jnp.int32).reshape(E // CB, CB)
    w = edge_attr.astype(jnp.float32)

    deg_k = _deg_kernel(E, N)
    edge_k = _edge_kernel(E, N, H)

    deg_part = deg_k(dst2, w)
    dis, g1 = _tc_h1_dis_g(x, W1, deg_part)

    s1 = edge_k(g1, src2, dst2, w)
    g2 = _tc_layer_out(s1, g1, dis, b1, W2, scale_out=True)

    s2 = edge_k(g2, src2, dst2, w)
    out = _tc_head(s2, g2, dis, b2, Wfc, bfc)
    return out


# speedup vs baseline: 1.0046x; 1.0046x over previous
import dataclasses
import functools

import jax
import jax.numpy as jnp
from jax import lax
from jax.experimental import pallas as pl
from jax.experimental.pallas import tpu as pltpu
from jax.experimental.pallas import tpu_sc as plsc

NC = 2
NS = 16
LANES = 16
CB = 80


def _sc_compiler_params():
    cp = pltpu.CompilerParams()
    cp = dataclasses.replace(
        cp, needs_layout_passes=False, use_tc_tiling_on_sc=False,
        internal_scratch_in_bytes=256 * 1024,
    )
    return cp


def _mesh():
    return plsc.VectorSubcoreMesh(
        core_axis_name="c", subcore_axis_name="s", num_cores=NC, num_subcores=NS
    )


def _zero_acc(zsrc, acc, sid, RB, TAIL, ZR):
    for t in range(RB // ZR):
        pltpu.sync_copy(zsrc, acc.at[pl.ds(sid * RB + t * ZR, ZR)])
    if TAIL:
        @pl.when(sid == NS - 1)
        def _():
            pltpu.sync_copy(zsrc.at[pl.ds(0, TAIL)], acc.at[pl.ds(NS * RB, TAIL)])


def _write_out(acc, out_hbm, cid, sid, RB, TAIL):
    pltpu.sync_copy(
        acc.at[pl.ds(sid * RB, RB)], out_hbm.at[cid, pl.ds(sid * RB, RB)]
    )
    if TAIL:
        @pl.when(sid == NS - 1)
        def _():
            pltpu.sync_copy(acc.at[pl.ds(NS * RB, TAIL)],
                            out_hbm.at[cid, pl.ds(NS * RB, TAIL)])


def _deg_kernel(E, N):
    NW = NC * NS
    EPW = E // NW
    NCH = EPW // CB
    RB = (N // NS) & ~7
    TAIL = N - RB * NS
    ZR = 208

    @functools.partial(
        pl.kernel,
        mesh=_mesh(),
        out_type=jax.ShapeDtypeStruct((NC, N, LANES), jnp.float32),
        compiler_params=_sc_compiler_params(),
        scratch_types=[
            pltpu.VMEM((NCH, CB), jnp.int32),
            pltpu.VMEM((EPW,), jnp.float32),
            pltpu.VMEM((CB, LANES), jnp.float32),
            pltpu.VMEM((CB, LANES), jnp.float32),
            pltpu.VMEM((ZR, LANES), jnp.float32),
            pltpu.VMEM_SHARED((N, LANES), jnp.float32),
            pltpu.SemaphoreType.DMA,
            pltpu.SemaphoreType.DMA,
        ],
    )
    def k(dst2_hbm, w_hbm, out_hbm, didx, wbuf, w0, w1, zsrc, acc, s0, s1):
        cid = lax.axis_index("c")
        sid = lax.axis_index("s")
        wid = cid * NS + sid

        @pl.loop(0, ZR)
        def _(r):
            zsrc[r, :] = jnp.zeros((LANES,), jnp.float32)

        _zero_acc(zsrc, acc, sid, RB, TAIL, ZR)
        pltpu.sync_copy(dst2_hbm.at[pl.ds(wid * NCH, NCH)], didx)
        pltpu.sync_copy(w_hbm.at[pl.ds(wid * EPW, EPW)], wbuf)
        plsc.subcore_barrier()

        wrows = (w0, w1)
        sems = (s0, s1)

        def build(j, wr):
            @plsc.parallel_loop(0, CB, unroll=8)
            def _(e):
                wv = plsc.load_gather(
                    wbuf, [jnp.full((LANES,), j * CB + e, jnp.int32)]
                )
                wr[e, :] = wv

        @pl.loop(0, NCH)
        def _(j):
            for b in range(2):
                @pl.when(j % 2 == b)
                def _():
                    @pl.when(j >= 2)
                    def _():
                        pltpu.make_async_copy(
                            wrows[b], acc.at[didx.at[j]], sems[b]
                        ).wait()
                    build(j, wrows[b])
                    pltpu.async_copy(
                        wrows[b], acc.at[didx.at[j]], sems[b], add=True
                    )

        for b in range(2):
            @pl.when(NCH > 2 - b)
            def _():
                pltpu.make_async_copy(
                    wrows[b], acc.at[didx.at[0]], sems[b]
                ).wait()

        plsc.subcore_barrier()
        _write_out(acc, out_hbm, cid, sid, RB, TAIL)

    return k


def _edge_kernel(E, N, D):
    NW = NC * NS
    EPW = E // NW
    NCH = EPW // CB
    RB = (N // NS) & ~7
    TAIL = N - RB * NS
    ZR = 208

    @functools.partial(
        pl.kernel,
        mesh=_mesh(),
        out_type=jax.ShapeDtypeStruct((NC, N, D), jnp.float32),
        compiler_params=_sc_compiler_params(),
        scratch_types=[
            pltpu.VMEM((NCH, CB), jnp.int32),
            pltpu.VMEM((NCH, CB), jnp.int32),
            pltpu.VMEM((EPW,), jnp.float32),
            pltpu.VMEM((CB, D), jnp.float32),
            pltpu.VMEM((CB, D), jnp.float32),
            pltpu.VMEM((CB, D), jnp.float32),
            pltpu.VMEM((ZR, D), jnp.float32),
            pltpu.VMEM_SHARED((N, D), jnp.float32),
            pltpu.SemaphoreType.DMA,
            pltpu.SemaphoreType.DMA,
            pltpu.SemaphoreType.DMA,
            pltpu.SemaphoreType.DMA,
            pltpu.SemaphoreType.DMA,
            pltpu.SemaphoreType.DMA,
        ],
    )
    def k(g_hbm, src2_hbm, dst2_hbm, w_hbm, out_hbm,
          sidx, didx, wbuf, r0, r1, r2, zsrc, acc, g0, g1, g2, s0, s1, s2):
        cid = lax.axis_index("c")
        sid = lax.axis_index("s")
        wid = cid * NS + sid

        @pl.loop(0, ZR)
        def _(r):
            for c4 in range(D // LANES):
                zsrc[r, pl.ds(c4 * LANES, LANES)] = jnp.zeros((LANES,), jnp.float32)

        _zero_acc(zsrc, acc, sid, RB, TAIL, ZR)
        pltpu.sync_copy(src2_hbm.at[pl.ds(wid * NCH, NCH)], sidx)
        pltpu.sync_copy(dst2_hbm.at[pl.ds(wid * NCH, NCH)], didx)
        pltpu.sync_copy(w_hbm.at[pl.ds(wid * EPW, EPW)], wbuf)
        plsc.subcore_barrier()

        rows = (r0, r1, r2)
        gsem = (g0, g1, g2)
        ssem = (s0, s1, s2)

        def compute(j, rr):
            @plsc.parallel_loop(0, CB, unroll=8)
            def _(e):
                wv = plsc.load_gather(
                    wbuf, [jnp.full((LANES,), j * CB + e, jnp.int32)]
                )
                for c4 in range(D // LANES):
                    sl = pl.ds(c4 * LANES, LANES)
                    rr[e, sl] = rr[e, sl] * wv

        pltpu.async_copy(g_hbm.at[sidx.at[0]], r0, g0)
        if NCH > 1:
            pltpu.async_copy(g_hbm.at[sidx.at[1]], r1, g1)

        @pl.loop(0, NCH)
        def _(j):
            for b in range(3):
                b2 = (b + 2) % 3

                @pl.when(j % 3 == b)
                def _():
                    @pl.when(j >= 1)
                    def _():
                        pltpu.make_async_copy(
                            rows[b2], acc.at[didx.at[j]], ssem[b2]
                        ).wait()

                    @pl.when(j + 2 < NCH)
                    def _():
                        pltpu.async_copy(
                            g_hbm.at[sidx.at[j + 2]], rows[b2], gsem[b2]
                        )

                    pltpu.make_async_copy(
                        g_hbm.at[sidx.at[j]], rows[b], gsem[b]
                    ).wait()
                    compute(j, rows[b])
                    pltpu.async_copy(
                        rows[b], acc.at[didx.at[j]], ssem[b], add=True
                    )

        pltpu.make_async_copy(
            rows[(NCH - 1) % 3], acc.at[didx.at[0]], ssem[(NCH - 1) % 3]
        ).wait()

        plsc.subcore_barrier()
        _write_out(acc, out_hbm, cid, sid, RB, TAIL)

    return k


_DOT = dict(preferred_element_type=jnp.float32, precision=lax.Precision.HIGHEST)


def _tc_h1(x, W1):
    N = x.shape[0]
    H = W1.shape[0]

    def body(x_ref, w_ref, o_ref):
        o_ref[...] = lax.dot_general(
            x_ref[...], w_ref[...], (((1,), (1,)), ((), ())), **_DOT
        )

    return pl.pallas_call(
        body, out_shape=jax.ShapeDtypeStruct((N, H), jnp.float32)
    )(x, W1)


def _tc_dis_g(deg_part, h):
    N, D = h.shape

    def body(dp_ref, h_ref, dis_ref, g_ref):
        deg = dp_ref[0] + dp_ref[1]
        dis = lax.rsqrt(deg[:, 0:1] + 1.0)
        dis_ref[...] = dis
        g_ref[...] = h_ref[...] * dis

    return pl.pallas_call(
        body,
        out_shape=(
            jax.ShapeDtypeStruct((N, 1), jnp.float32),
            jax.ShapeDtypeStruct((N, D), jnp.float32),
        ),
    )(deg_part, h)


def _tc_layer_out(s_part, g, dis, b, W_next, scale_out):
    N, D = g.shape
    H = W_next.shape[0]

    def body(sp_ref, g_ref, dis_ref, b_ref, w_ref, o_ref):
        s = sp_ref[0] + sp_ref[1] + g_ref[...]
        xx = jax.nn.relu(dis_ref[...] * s + b_ref[...])
        h = lax.dot_general(xx, w_ref[...], (((1,), (1,)), ((), ())), **_DOT)
        if scale_out:
            h = h * dis_ref[...]
        o_ref[...] = h

    return pl.pallas_call(
        body, out_shape=jax.ShapeDtypeStruct((N, H), jnp.float32)
    )(s_part, g, dis, b.reshape(1, D), W_next)


def _tc_head(s_part, g, dis, b, Wfc, bfc):
    N, D = g.shape
    O = Wfc.shape[0]

    def body(sp_ref, g_ref, dis_ref, b_ref, w_ref, bfc_ref, o_ref):
        s = sp_ref[0] + sp_ref[1] + g_ref[...]
        xx = jax.nn.relu(dis_ref[...] * s + b_ref[...])
        o_ref[...] = (
            lax.dot_general(xx, w_ref[...], (((1,), (1,)), ((), ())), **_DOT)
            + bfc_ref[...]
        )

    return pl.pallas_call(
        body, out_shape=jax.ShapeDtypeStruct((N, O), jnp.float32)
    )(s_part, g, dis, b.reshape(1, D), Wfc, bfc.reshape(1, O))


def kernel(x, edge_index, edge_attr, W1, b1, W2, b2, Wfc, bfc):
    N = x.shape[0]
    E = edge_index.shape[1]
    H = W1.shape[0]

    src2 = edge_index[0].astype(jnp.int32).reshape(E // CB, CB)
    dst2 = edge_index[1].astype(jnp.int32).reshape(E // CB, CB)
    w = edge_attr.astype(jnp.float32)

    deg_k = _deg_kernel(E, N)
    edge_k = _edge_kernel(E, N, H)

    h1 = _tc_h1(x, W1)
    deg_part = deg_k(dst2, w)
    dis, g1 = _tc_dis_g(deg_part, h1)

    s1 = edge_k(g1, src2, dst2, w)
    g2 = _tc_layer_out(s1, g1, dis, b1, W2, scale_out=True)

    s2 = edge_k(g2, src2, dst2, w)
    out = _tc_head(s2, g2, dis, b2, Wfc, bfc)
    return out
